# baseline (speedup 1.0000x reference)
import numpy as np
import jax
import jax.numpy as jnp
from jax.experimental import pallas as pl

_N = 360

_DN_T = (((1,), (1,)), ((), ()))


def _fused(x_ref, w1_ref, b1_ref, w2_ref, b2_ref, w3_ref, b3_ref,
           wg_ref, bg_ref, wm1_ref, bm1_ref, wm2_ref, bm2_ref, out_ref):
    f32 = jnp.float32
    hi = jax.lax.Precision.HIGHEST

    def mm_t(v, w_ref):
        return jax.lax.dot_general(v, w_ref[:], _DN_T,
                                   preferred_element_type=f32, precision=hi)

    i = jax.lax.broadcasted_iota(jnp.int32, (_N, 1), 0)
    fi = i.astype(f32)

    ang = fi * np.float32(2.0 * np.pi / (_N - 1))
    scan = jnp.transpose(x_ref[0:1, 0:_N])
    nx = scan * jnp.cos(ang)
    ny = scan * jnp.sin(ang)

    end = (i == 0) | (i == (_N - 1))
    dis = jnp.where(end, np.float32(1.0 / np.sqrt(2.0)),
                    np.float32(1.0 / np.sqrt(3.0)))
    cd = dis * dis
    cl = jnp.where(i == 0, 0.0, dis * jnp.roll(dis, 1, axis=0))
    cu = jnp.where(i == (_N - 1), 0.0, dis * jnp.roll(dis, -1, axis=0))

    def agg(v):
        return cd * v + cl * jnp.roll(v, 1, axis=0) + cu * jnp.roll(v, -1, axis=0)

    w1 = w1_ref[:]
    nodes = jnp.concatenate([nx, ny], axis=1)
    xw = jax.lax.dot_general(nodes, w1, _DN_T,
                             preferred_element_type=f32, precision=hi)
    h = jnp.maximum(agg(xw) + b1_ref[:], 0.0)

    h = jnp.maximum(agg(mm_t(h, w2_ref)) + b2_ref[:], 0.0)
    h = jnp.maximum(agg(mm_t(h, w3_ref)) + b3_ref[:], 0.0)

    g = jnp.mean(h, axis=0, keepdims=True)
    c = mm_t(g, wg_ref) + bg_ref[:]
    m = jnp.maximum(mm_t(c, wm1_ref) + bm1_ref[:], 0.0)
    out_ref[:] = mm_t(m, wm2_ref) + bm2_ref[:]


@jax.jit
def _run(x, W1, b1, W2, b2, W3, b3, Wg, bg, Wm1, bm1, Wm2, bm2):
    out = pl.pallas_call(
        _fused,
        out_shape=jax.ShapeDtypeStruct((1, 200), jnp.float32),
    )(x, W1, b1, W2, b2, W3, b3, Wg, bg, Wm1, bm1, Wm2, bm2)
    return out.reshape(1, 10, 10, 2)


def kernel(x, W1, b1, W2, b2, W3, b3, Wg, bg, Wm1, bm1, Wm2, bm2):
    return _run(x, W1, b1, W2, b2, W3, b3, Wg, bg, Wm1, bm1, Wm2, bm2)


# speedup vs baseline: 1.2181x; 1.2181x over previous
import numpy as np
import jax
import jax.numpy as jnp
from jax.experimental import pallas as pl

_N = 360

_DN_T = (((1,), (1,)), ((), ()))


def _fused(scan_ref, w1_ref, b1_ref, w2_ref, b2_ref, w3_ref, b3_ref,
           wg_ref, bg_ref, wm1_ref, bm1_ref, wm2_ref, bm2_ref, out_ref):
    f32 = jnp.float32
    hi = jax.lax.Precision.DEFAULT

    def mm_t(v, w_ref):
        return jax.lax.dot_general(v, w_ref[:], _DN_T,
                                   preferred_element_type=f32, precision=hi)

    i = jax.lax.broadcasted_iota(jnp.int32, (_N, 1), 0)
    fi = i.astype(f32)

    ang = fi * np.float32(2.0 * np.pi / (_N - 1))
    scan = scan_ref[:]
    nx = scan * jnp.cos(ang)
    ny = scan * jnp.sin(ang)

    end = (i == 0) | (i == (_N - 1))
    dis = jnp.where(end, np.float32(1.0 / np.sqrt(2.0)),
                    np.float32(1.0 / np.sqrt(3.0)))
    cd = dis * dis
    cl = jnp.where(i == 0, 0.0, dis * jnp.roll(dis, 1, axis=0))
    cu = jnp.where(i == (_N - 1), 0.0, dis * jnp.roll(dis, -1, axis=0))

    def agg(v):
        return cd * v + cl * jnp.roll(v, 1, axis=0) + cu * jnp.roll(v, -1, axis=0)

    w1 = w1_ref[:]
    nodes = jnp.concatenate([nx, ny], axis=1)
    xw = jax.lax.dot_general(nodes, w1, _DN_T,
                             preferred_element_type=f32, precision=hi)
    h = jnp.maximum(agg(xw) + b1_ref[:], 0.0)

    h = jnp.maximum(agg(mm_t(h, w2_ref)) + b2_ref[:], 0.0)
    h = jnp.maximum(agg(mm_t(h, w3_ref)) + b3_ref[:], 0.0)

    g = jnp.mean(h, axis=0, keepdims=True)
    c = mm_t(g, wg_ref) + bg_ref[:]
    m = jnp.maximum(mm_t(c, wm1_ref) + bm1_ref[:], 0.0)
    out_ref[:] = mm_t(m, wm2_ref) + bm2_ref[:]


@jax.jit
def _run(x, W1, b1, W2, b2, W3, b3, Wg, bg, Wm1, bm1, Wm2, bm2):
    scan = x[0, :_N].reshape(_N, 1)
    out = pl.pallas_call(
        _fused,
        out_shape=jax.ShapeDtypeStruct((1, 200), jnp.float32),
    )(scan, W1, b1, W2, b2, W3, b3, Wg, bg, Wm1, bm1, Wm2, bm2)
    return out.reshape(1, 10, 10, 2)


def kernel(x, W1, b1, W2, b2, W3, b3, Wg, bg, Wm1, bm1, Wm2, bm2):
    return _run(x, W1, b1, W2, b2, W3, b3, Wg, bg, Wm1, bm1, Wm2, bm2)


# speedup vs baseline: 1.2641x; 1.0378x over previous
import numpy as np
import jax
import jax.numpy as jnp
from jax.experimental import pallas as pl
from jax.experimental.pallas import tpu as pltpu

_N = 360

_DN_T = (((1,), (1,)), ((), ()))

_WM1_CHUNKS = 4
_WM2_CHUNKS = 1


def _fused(scan_ref, w1_ref, b1_ref, w2_ref, b2_ref, w3_ref, b3_ref,
           bg_ref, bm1_ref, bm2_ref, wg_hbm, wm1_hbm, wm2_hbm,
           out_ref, wg_s, wm1_s, wm2_s, sems):
    f32 = jnp.float32

    def mm_t(v, w):
        return jax.lax.dot_general(v, w, _DN_T, preferred_element_type=f32)

    cp_g = pltpu.make_async_copy(wg_hbm, wg_s, sems.at[0])
    cp_g.start()
    cp_m1 = []
    for k in range(_WM1_CHUNKS):
        r = 1024 // _WM1_CHUNKS
        cp = pltpu.make_async_copy(wm1_hbm.at[pl.ds(k * r, r), :],
                                   wm1_s.at[pl.ds(k * r, r), :], sems.at[1 + k])
        cp.start()
        cp_m1.append(cp)
    cp_m2 = []
    for k in range(_WM2_CHUNKS):
        r = 200 // _WM2_CHUNKS
        cp = pltpu.make_async_copy(wm2_hbm.at[pl.ds(k * r, r), :],
                                   wm2_s.at[pl.ds(k * r, r), :],
                                   sems.at[1 + _WM1_CHUNKS + k])
        cp.start()
        cp_m2.append(cp)

    i = jax.lax.broadcasted_iota(jnp.int32, (_N, 1), 0)
    fi = i.astype(f32)

    ang = fi * np.float32(2.0 * np.pi / (_N - 1))
    scan = scan_ref[:]
    nx = scan * jnp.cos(ang)
    ny = scan * jnp.sin(ang)

    end = (i == 0) | (i == (_N - 1))
    dis = jnp.where(end, np.float32(1.0 / np.sqrt(2.0)),
                    np.float32(1.0 / np.sqrt(3.0)))
    cd = dis * dis
    cl = jnp.where(i == 0, 0.0, dis * jnp.roll(dis, 1, axis=0))
    cu = jnp.where(i == (_N - 1), 0.0, dis * jnp.roll(dis, -1, axis=0))

    def agg(v):
        return cd * v + cl * jnp.roll(v, 1, axis=0) + cu * jnp.roll(v, -1, axis=0)

    nodes = jnp.concatenate([nx, ny], axis=1)
    xw = mm_t(nodes, w1_ref[:])
    h = jnp.maximum(agg(xw) + b1_ref[:], 0.0)

    h = jnp.maximum(agg(mm_t(h, w2_ref[:])) + b2_ref[:], 0.0)
    h = jnp.maximum(agg(mm_t(h, w3_ref[:])) + b3_ref[:], 0.0)

    g = jnp.mean(h, axis=0, keepdims=True)
    cp_g.wait()
    c = mm_t(g, wg_s[:]) + bg_ref[:]
    for cp in cp_m1:
        cp.wait()
    m = jnp.maximum(mm_t(c, wm1_s[:]) + bm1_ref[:], 0.0)
    for cp in cp_m2:
        cp.wait()
    out_ref[:] = mm_t(m, wm2_s[:]) + bm2_ref[:]


@jax.jit
def _run(x, W1, b1, W2, b2, W3, b3, Wg, bg, Wm1, bm1, Wm2, bm2):
    scan = x[0, :_N].reshape(_N, 1)
    vmem = pl.BlockSpec(memory_space=pltpu.MemorySpace.VMEM)
    hbm = pl.BlockSpec(memory_space=pltpu.MemorySpace.HBM)
    out = pl.pallas_call(
        _fused,
        out_shape=jax.ShapeDtypeStruct((1, 200), jnp.float32),
        in_specs=[vmem] * 10 + [hbm] * 3,
        out_specs=vmem,
        scratch_shapes=[
            pltpu.VMEM((512, 64), jnp.float32),
            pltpu.VMEM((1024, 512), jnp.float32),
            pltpu.VMEM((200, 1024), jnp.float32),
            pltpu.SemaphoreType.DMA((1 + _WM1_CHUNKS + _WM2_CHUNKS,)),
        ],
    )(scan, W1, b1, W2, b2, W3, b3, bg, bm1, bm2, Wg, Wm1, Wm2)
    return out.reshape(1, 10, 10, 2)


def kernel(x, W1, b1, W2, b2, W3, b3, Wg, bg, Wm1, bm1, Wm2, bm2):
    return _run(x, W1, b1, W2, b2, W3, b3, Wg, bg, Wm1, bm1, Wm2, bm2)


# speedup vs baseline: 1.4340x; 1.1344x over previous
import numpy as np
import jax
import jax.numpy as jnp
from jax.experimental import pallas as pl
from jax.experimental.pallas import tpu as pltpu

_N = 360

_DN_T = (((1,), (1,)), ((), ()))

_WM1_CHUNKS = 8
_WM2_ROWS = (104, 96)


def _fused(x_ref, w1_ref, b1_ref, w2_ref, b2_ref, w3_ref, b3_ref,
           bg_ref, bm1_ref, bm2_ref, wg_hbm, wm1_hbm, wm2_hbm,
           out_ref, wg_s, wm1_s, wm2_s, sems):
    f32 = jnp.float32

    def mm_t(v, w):
        return jax.lax.dot_general(v, w, _DN_T, preferred_element_type=f32)

    cp_g = pltpu.make_async_copy(wg_hbm, wg_s, sems.at[0])
    cp_g.start()
    cp_m1 = []
    for k in range(_WM1_CHUNKS):
        r = 1024 // _WM1_CHUNKS
        cp = pltpu.make_async_copy(wm1_hbm.at[pl.ds(k * r, r), :],
                                   wm1_s.at[pl.ds(k * r, r), :], sems.at[1 + k])
        cp.start()
        cp_m1.append(cp)
    cp_m2 = []
    base = 0
    for k, r in enumerate(_WM2_ROWS):
        cp = pltpu.make_async_copy(wm2_hbm.at[pl.ds(base, r), :],
                                   wm2_s.at[pl.ds(base, r), :],
                                   sems.at[1 + _WM1_CHUNKS + k])
        cp.start()
        cp_m2.append(cp)
        base += r

    i = jax.lax.broadcasted_iota(jnp.int32, (_N, 1), 0)
    fi = i.astype(f32)

    ang = fi * np.float32(2.0 * np.pi / (_N - 1))
    scan = jnp.transpose(x_ref[0:1, 0:_N])
    nx = scan * jnp.cos(ang)
    ny = scan * jnp.sin(ang)

    end = (i == 0) | (i == (_N - 1))
    dis = jnp.where(end, np.float32(1.0 / np.sqrt(2.0)),
                    np.float32(1.0 / np.sqrt(3.0)))
    cd = dis * dis
    cl = jnp.where(i == 0, 0.0, dis * jnp.roll(dis, 1, axis=0))
    cu = jnp.where(i == (_N - 1), 0.0, dis * jnp.roll(dis, -1, axis=0))

    def agg(v):
        return cd * v + cl * jnp.roll(v, 1, axis=0) + cu * jnp.roll(v, -1, axis=0)

    nodes = jnp.concatenate([nx, ny], axis=1)
    xw = mm_t(nodes, w1_ref[:])
    h = jnp.maximum(agg(xw) + b1_ref[:], 0.0)

    h = jnp.maximum(agg(mm_t(h, w2_ref[:])) + b2_ref[:], 0.0)
    h = jnp.maximum(agg(mm_t(h, w3_ref[:])) + b3_ref[:], 0.0)

    g = jnp.mean(h, axis=0, keepdims=True)
    cp_g.wait()
    c = mm_t(g, wg_s[:]) + bg_ref[:]
    for cp in cp_m1:
        cp.wait()
    m = jnp.maximum(mm_t(c, wm1_s[:]) + bm1_ref[:], 0.0)
    for cp in cp_m2:
        cp.wait()
    out_ref[:] = mm_t(m, wm2_s[:]) + bm2_ref[:]


@jax.jit
def _run(x, W1, b1, W2, b2, W3, b3, Wg, bg, Wm1, bm1, Wm2, bm2):
    vmem = pl.BlockSpec(memory_space=pltpu.MemorySpace.VMEM)
    hbm = pl.BlockSpec(memory_space=pltpu.MemorySpace.HBM)
    out = pl.pallas_call(
        _fused,
        out_shape=jax.ShapeDtypeStruct((1, 200), jnp.float32),
        in_specs=[vmem] * 10 + [hbm] * 3,
        out_specs=vmem,
        scratch_shapes=[
            pltpu.VMEM((512, 64), jnp.float32),
            pltpu.VMEM((1024, 512), jnp.float32),
            pltpu.VMEM((200, 1024), jnp.float32),
            pltpu.SemaphoreType.DMA((1 + _WM1_CHUNKS + len(_WM2_ROWS),)),
        ],
    )(x, W1, b1, W2, b2, W3, b3, bg, bm1, bm2, Wg, Wm1, Wm2)
    return out.reshape(1, 10, 10, 2)


def kernel(x, W1, b1, W2, b2, W3, b3, Wg, bg, Wm1, bm1, Wm2, bm2):
    return _run(x, W1, b1, W2, b2, W3, b3, Wg, bg, Wm1, bm1, Wm2, bm2)


# speedup vs baseline: 1.5450x; 1.0774x over previous
import numpy as np
import jax
import jax.numpy as jnp
from jax.experimental import pallas as pl
from jax.experimental.pallas import tpu as pltpu

_N = 360

_DN_T = (((1,), (1,)), ((), ()))

_WM1_CHUNKS = 16
_WM2_ROWS = (104, 96)


def _fused(x_ref, w1_ref, b1_ref, w2_ref, b2_ref, w3_ref, b3_ref,
           bg_ref, bm1_ref, bm2_ref, wg_hbm, wm1_hbm, wm2_hbm,
           out_ref, wg_s, wm1_s, wm2_s, sems):
    f32 = jnp.float32

    def mm_t(v, w):
        return jax.lax.dot_general(v, w, _DN_T, preferred_element_type=f32)

    cp_g = pltpu.make_async_copy(wg_hbm, wg_s, sems.at[0])
    cp_g.start()
    cp_m1 = []
    for k in range(_WM1_CHUNKS):
        r = 1024 // _WM1_CHUNKS
        cp = pltpu.make_async_copy(wm1_hbm.at[pl.ds(k * r, r), :],
                                   wm1_s.at[pl.ds(k * r, r), :], sems.at[1 + k])
        cp.start()
        cp_m1.append(cp)
    cp_m2 = []
    base = 0
    for k, r in enumerate(_WM2_ROWS):
        cp = pltpu.make_async_copy(wm2_hbm.at[pl.ds(base, r), :],
                                   wm2_s.at[pl.ds(base, r), :],
                                   sems.at[1 + _WM1_CHUNKS + k])
        cp.start()
        cp_m2.append(cp)
        base += r

    i = jax.lax.broadcasted_iota(jnp.int32, (_N, 1), 0)
    fi = i.astype(f32)

    ang = fi * np.float32(2.0 * np.pi / (_N - 1))
    scan = jnp.transpose(x_ref[0:1, 0:_N])
    nx = scan * jnp.cos(ang)
    ny = scan * jnp.sin(ang)

    end = (i == 0) | (i == (_N - 1))
    dis = jnp.where(end, np.float32(1.0 / np.sqrt(2.0)),
                    np.float32(1.0 / np.sqrt(3.0)))
    cd = dis * dis
    cl = jnp.where(i == 0, 0.0, dis * jnp.roll(dis, 1, axis=0))
    cu = jnp.where(i == (_N - 1), 0.0, dis * jnp.roll(dis, -1, axis=0))

    def agg(v):
        return cd * v + cl * jnp.roll(v, 1, axis=0) + cu * jnp.roll(v, -1, axis=0)

    nodes = jnp.concatenate([nx, ny], axis=1)
    xw = mm_t(nodes, w1_ref[:])
    h = jnp.maximum(agg(xw) + b1_ref[:], 0.0)

    h = jnp.maximum(agg(mm_t(h, w2_ref[:])) + b2_ref[:], 0.0)
    h = jnp.maximum(agg(mm_t(h, w3_ref[:])) + b3_ref[:], 0.0)

    g = jnp.mean(h, axis=0, keepdims=True)
    cp_g.wait()
    c = mm_t(g, wg_s[:]) + bg_ref[:]
    for cp in cp_m1:
        cp.wait()
    m = jnp.maximum(mm_t(c, wm1_s[:]) + bm1_ref[:], 0.0)
    for cp in cp_m2:
        cp.wait()
    row = mm_t(m, wm2_s[:]) + bm2_ref[:]
    out_ref[:] = row.reshape(1, 10, 10, 2)


@jax.jit
def _run(x, W1, b1, W2, b2, W3, b3, Wg, bg, Wm1, bm1, Wm2, bm2):
    vmem = pl.BlockSpec(memory_space=pltpu.MemorySpace.VMEM)
    hbm = pl.BlockSpec(memory_space=pltpu.MemorySpace.HBM)
    out = pl.pallas_call(
        _fused,
        out_shape=jax.ShapeDtypeStruct((1, 10, 10, 2), jnp.float32),
        in_specs=[vmem] * 10 + [hbm] * 3,
        out_specs=vmem,
        scratch_shapes=[
            pltpu.VMEM((512, 64), jnp.float32),
            pltpu.VMEM((1024, 512), jnp.float32),
            pltpu.VMEM((200, 1024), jnp.float32),
            pltpu.SemaphoreType.DMA((1 + _WM1_CHUNKS + len(_WM2_ROWS),)),
        ],
    )(x, W1, b1, W2, b2, W3, b3, bg, bm1, bm2, Wg, Wm1, Wm2)
    return out


def kernel(x, W1, b1, W2, b2, W3, b3, Wg, bg, Wm1, bm1, Wm2, bm2):
    return _run(x, W1, b1, W2, b2, W3, b3, Wg, bg, Wm1, bm1, Wm2, bm2)
